# Initial kernel scaffold; baseline (speedup 1.0000x reference)
import functools

import jax
import jax.numpy as jnp
from jax import lax
from jax.experimental import pallas as pl
from jax.experimental.pallas import tpu as pltpu
from jax.experimental.pallas import tpu_sc as plsc

_CH = 128
_NC = 2
_NS = 16
_NW = _NC * _NS


@functools.lru_cache(maxsize=None)
def _make_sc_aggregate(n_pad: int, cpt: int, d: int):
    rows_per_tile = n_pad // _NS
    mesh = plsc.VectorSubcoreMesh(core_axis_name="c", subcore_axis_name="s")

    @functools.partial(
        pl.kernel,
        out_type=jax.ShapeDtypeStruct((_NC, n_pad, d), jnp.float32),
        mesh=mesh,
        scratch_types=[
            pltpu.VMEM((cpt, _CH), jnp.int32),
            pltpu.VMEM((cpt, _CH), jnp.int32),
            pltpu.VMEM((_CH, d), jnp.float32),
            pltpu.VMEM((_CH, d), jnp.float32),
            pltpu.VMEM_SHARED((n_pad, d), jnp.float32),
            pltpu.SemaphoreType.DMA,
            pltpu.SemaphoreType.DMA,
        ],
    )
    def sc_aggregate(h_hbm, src_hbm, dst_hbm, out_hbm,
                     src_v, dst_v, buf_a, buf_b, aggr_sh, sem_a, sem_b):
        c = lax.axis_index("c")
        s = lax.axis_index("s")
        wid = s * _NC + c

        zero16 = jnp.zeros((16,), jnp.float32)

        def zero_row(i, carry):
            for j in range(d // 16):
                buf_a[i, pl.ds(j * 16, 16)] = zero16
            return carry

        lax.fori_loop(0, _CH, zero_row, 0)
        for k in range(rows_per_tile // _CH):
            pltpu.sync_copy(buf_a, aggr_sh.at[pl.ds(s * rows_per_tile + k * _CH, _CH)])
        plsc.subcore_barrier()

        pltpu.sync_copy(src_hbm.at[pl.ds(wid * cpt, cpt)], src_v)
        pltpu.sync_copy(dst_hbm.at[pl.ds(wid * cpt, cpt)], dst_v)

        def body(i, carry):
            j0 = 2 * i
            j1 = j0 + 1
            cp_a = pltpu.async_copy(h_hbm.at[src_v.at[j0]], buf_a, sem_a)
            cp_b = pltpu.async_copy(h_hbm.at[src_v.at[j1]], buf_b, sem_b)
            cp_a.wait()
            pltpu.sync_copy(buf_a, aggr_sh.at[dst_v.at[j0]], add=True)
            cp_b.wait()
            pltpu.sync_copy(buf_b, aggr_sh.at[dst_v.at[j1]], add=True)
            return carry

        lax.fori_loop(0, cpt // 2, body, 0)
        plsc.subcore_barrier()

        pltpu.sync_copy(aggr_sh.at[pl.ds(s * rows_per_tile, rows_per_tile)],
                        out_hbm.at[c].at[pl.ds(s * rows_per_tile, rows_per_tile)])

    return sc_aggregate


def _tc_dense(parts, h, w_rel_t, w_root_t, b_2d):
    n_pad, d = h.shape
    blk = 1024

    def body(p_ref, h_ref, wr_ref, wo_ref, b_ref, o_ref):
        aggr = p_ref[0] + p_ref[1]
        z = jnp.dot(aggr, wr_ref[...], preferred_element_type=jnp.float32)
        z = z + jnp.dot(h_ref[...], wo_ref[...], preferred_element_type=jnp.float32)
        z = z + b_ref[...]
        o_ref[...] = jnp.where(z > 0, z, jnp.expm1(z))

    return pl.pallas_call(
        body,
        grid=(n_pad // blk,),
        in_specs=[
            pl.BlockSpec((_NC, blk, d), lambda i: (0, i, 0)),
            pl.BlockSpec((blk, d), lambda i: (i, 0)),
            pl.BlockSpec((d, d), lambda i: (0, 0)),
            pl.BlockSpec((d, d), lambda i: (0, 0)),
            pl.BlockSpec((1, d), lambda i: (0, 0)),
        ],
        out_specs=pl.BlockSpec((blk, d), lambda i: (i, 0)),
        out_shape=jax.ShapeDtypeStruct((n_pad, d), jnp.float32),
    )(parts, h, w_rel_t, w_root_t, b_2d)


def kernel(x, edge_index, W1_rel, b1_rel, W1_root, W2_rel, b2_rel, W2_root,
           W3_rel, b3_rel, W3_root):
    n, d = x.shape
    e = edge_index.shape[1]

    cpt = -(-e // (_NW * _CH))
    cpt += cpt % 2
    e_pad = cpt * _NW * _CH
    n_pad = -(-(n + 1) // (_NS * _CH)) * (_NS * _CH)

    ei = edge_index.astype(jnp.int32)
    fill = jnp.full((e_pad - e,), n, jnp.int32)
    src2d = jnp.concatenate([ei[0], fill]).reshape(-1, _CH)
    dst2d = jnp.concatenate([ei[1], fill]).reshape(-1, _CH)

    h = jnp.zeros((n_pad, d), jnp.float32).at[:n].set(x)
    sc_aggregate = _make_sc_aggregate(n_pad, cpt, d)

    for w_rel, b_rel, w_root in ((W1_rel, b1_rel, W1_root),
                                 (W2_rel, b2_rel, W2_root),
                                 (W3_rel, b3_rel, W3_root)):
        parts = sc_aggregate(h, src2d, dst2d)
        h = _tc_dense(parts, h, w_rel.T, w_root.T, b_rel.reshape(1, d))
    return h[:n]


# speedup vs baseline: 3.2026x; 3.2026x over previous
import functools

import jax
import jax.numpy as jnp
from jax import lax
from jax.experimental import pallas as pl
from jax.experimental.pallas import tpu as pltpu
from jax.experimental.pallas import tpu_sc as plsc

_CH = 128
_NC = 2
_NS = 16
_NW = _NC * _NS


@functools.lru_cache(maxsize=None)
def _make_sc_aggregate(n_pad: int, cpt: int, d: int):
    rows_per_tile = n_pad // _NS
    mesh = plsc.VectorSubcoreMesh(core_axis_name="c", subcore_axis_name="s")

    @functools.partial(
        pl.kernel,
        out_type=jax.ShapeDtypeStruct((_NC, n_pad, d), jnp.float32),
        mesh=mesh,
        scratch_types=[
            pltpu.VMEM((cpt // 2, _CH), jnp.int32),
            pltpu.VMEM((cpt // 2, _CH), jnp.int32),
            pltpu.VMEM((_CH, d), jnp.float32),
            pltpu.VMEM((_CH, d), jnp.float32),
            pltpu.VMEM_SHARED((n_pad, d), jnp.float32),
            pltpu.SemaphoreType.DMA,
            pltpu.SemaphoreType.DMA,
        ],
    )
    def sc_aggregate(h_hbm, src_hbm, dst_hbm, out_hbm,
                     src_v, dst_v, buf_a, buf_b, aggr_sh, sem_a, sem_b):
        c = lax.axis_index("c")
        s = lax.axis_index("s")
        wid = s * _NC + c

        zero16 = jnp.zeros((16,), jnp.float32)

        def zero_row(i, carry):
            for j in range(d // 16):
                buf_a[i, pl.ds(j * 16, 16)] = zero16
            return carry

        lax.fori_loop(0, _CH, zero_row, 0)
        for k in range(rows_per_tile // _CH):
            pltpu.sync_copy(buf_a, aggr_sh.at[pl.ds(s * rows_per_tile + k * _CH, _CH)])
        plsc.subcore_barrier()

        cpp = cpt // 2

        def body(i, carry):
            j0 = 2 * i
            j1 = j0 + 1
            cp_a = pltpu.async_copy(h_hbm.at[src_v.at[j0]], buf_a, sem_a)
            cp_b = pltpu.async_copy(h_hbm.at[src_v.at[j1]], buf_b, sem_b)
            cp_a.wait()
            pltpu.sync_copy(buf_a, aggr_sh.at[dst_v.at[j0]], add=True)
            cp_b.wait()
            pltpu.sync_copy(buf_b, aggr_sh.at[dst_v.at[j1]], add=True)
            return carry

        for phase in range(2):
            pltpu.sync_copy(src_hbm.at[pl.ds(wid * cpt + phase * cpp, cpp)], src_v)
            pltpu.sync_copy(dst_hbm.at[pl.ds(wid * cpt + phase * cpp, cpp)], dst_v)
            lax.fori_loop(0, cpp // 2, body, 0)
        plsc.subcore_barrier()

        pltpu.sync_copy(aggr_sh.at[pl.ds(s * rows_per_tile, rows_per_tile)],
                        out_hbm.at[c].at[pl.ds(s * rows_per_tile, rows_per_tile)])

    return sc_aggregate


def _tc_dense(parts, h, w_rel_t, w_root_t, b_2d):
    n_pad, d = h.shape
    blk = 1024

    def body(p_ref, h_ref, wr_ref, wo_ref, b_ref, o_ref):
        aggr = p_ref[0] + p_ref[1]
        z = jnp.dot(aggr, wr_ref[...], preferred_element_type=jnp.float32)
        z = z + jnp.dot(h_ref[...], wo_ref[...], preferred_element_type=jnp.float32)
        z = z + b_ref[...]
        o_ref[...] = jnp.where(z > 0, z, jnp.exp(jnp.minimum(z, 0.0)) - 1.0)

    return pl.pallas_call(
        body,
        grid=(n_pad // blk,),
        in_specs=[
            pl.BlockSpec((_NC, blk, d), lambda i: (0, i, 0)),
            pl.BlockSpec((blk, d), lambda i: (i, 0)),
            pl.BlockSpec((d, d), lambda i: (0, 0)),
            pl.BlockSpec((d, d), lambda i: (0, 0)),
            pl.BlockSpec((1, d), lambda i: (0, 0)),
        ],
        out_specs=pl.BlockSpec((blk, d), lambda i: (i, 0)),
        out_shape=jax.ShapeDtypeStruct((n_pad, d), jnp.float32),
    )(parts, h, w_rel_t, w_root_t, b_2d)


def kernel(x, edge_index, W1_rel, b1_rel, W1_root, W2_rel, b2_rel, W2_root,
           W3_rel, b3_rel, W3_root):
    n, d = x.shape
    e = edge_index.shape[1]

    cpt = -(-e // (_NW * _CH * 4)) * 4
    e_pad = cpt * _NW * _CH
    n_pad = -(-(n + 1) // (_NS * _CH)) * (_NS * _CH)

    ei = edge_index.astype(jnp.int32)
    fill = jnp.full((e_pad - e,), n, jnp.int32)
    src2d = jnp.concatenate([ei[0], fill]).reshape(-1, _CH)
    dst2d = jnp.concatenate([ei[1], fill]).reshape(-1, _CH)

    h = jnp.zeros((n_pad, d), jnp.float32).at[:n].set(x)
    sc_aggregate = _make_sc_aggregate(n_pad, cpt, d)

    for w_rel, b_rel, w_root in ((W1_rel, b1_rel, W1_root),
                                 (W2_rel, b2_rel, W2_root),
                                 (W3_rel, b3_rel, W3_root)):
        parts = sc_aggregate(h, src2d, dst2d)
        h = _tc_dense(parts, h, w_rel.T, w_root.T, b_rel.reshape(1, d))
    return h[:n]


# speedup vs baseline: 9.5829x; 2.9922x over previous
import functools

import jax
import jax.numpy as jnp
from jax import lax
from jax.experimental import pallas as pl
from jax.experimental.pallas import tpu as pltpu
from jax.experimental.pallas import tpu_sc as plsc

_CH = 128
_NC = 2
_NS = 16
_NW = _NC * _NS


@functools.lru_cache(maxsize=None)
def _make_sc_aggregate(n_pad: int, cpt: int, d: int):
    rows_per_tile = n_pad // _NS
    mesh = plsc.VectorSubcoreMesh(core_axis_name="c", subcore_axis_name="s")

    @functools.partial(
        pl.kernel,
        out_type=jax.ShapeDtypeStruct((_NC, n_pad, d), jnp.float32),
        mesh=mesh,
        scratch_types=[
            pltpu.VMEM((cpt // 2, _CH), jnp.int32),
            pltpu.VMEM((cpt // 2, _CH), jnp.int32),
            pltpu.VMEM((_CH, d), jnp.float32),
            pltpu.VMEM((_CH, d), jnp.float32),
            pltpu.VMEM_SHARED((n_pad, d), jnp.float32),
            pltpu.SemaphoreType.DMA,
            pltpu.SemaphoreType.DMA,
        ],
    )
    def sc_aggregate(h_hbm, src_hbm, dst_hbm, out_hbm,
                     src_v, dst_v, buf_a, buf_b, aggr_sh, sem_a, sem_b):
        c = lax.axis_index("c")
        s = lax.axis_index("s")
        wid = s * _NC + c

        zero16 = jnp.zeros((16,), jnp.float32)

        def zero_row(i, carry):
            for j in range(d // 16):
                buf_a[i, pl.ds(j * 16, 16)] = zero16
            return carry

        lax.fori_loop(0, _CH, zero_row, 0)
        for k in range(rows_per_tile // _CH):
            pltpu.sync_copy(buf_a, aggr_sh.at[pl.ds(s * rows_per_tile + k * _CH, _CH)])
        plsc.subcore_barrier()

        cpp = cpt // 2

        def body(i, carry):
            j0 = 2 * i
            j1 = j0 + 1
            cp_a = pltpu.async_copy(h_hbm.at[src_v.at[j0]], buf_a, sem_a)
            cp_b = pltpu.async_copy(h_hbm.at[src_v.at[j1]], buf_b, sem_b)
            cp_a.wait()
            pltpu.sync_copy(buf_a, aggr_sh.at[dst_v.at[j0]], add=True)
            cp_b.wait()
            pltpu.sync_copy(buf_b, aggr_sh.at[dst_v.at[j1]], add=True)
            return carry

        for phase in range(2):
            pltpu.sync_copy(src_hbm.at[pl.ds(wid * cpt + phase * cpp, cpp)], src_v)
            pltpu.sync_copy(dst_hbm.at[pl.ds(wid * cpt + phase * cpp, cpp)], dst_v)
            lax.fori_loop(0, cpp // 2, body, 0)
        plsc.subcore_barrier()

        pltpu.sync_copy(aggr_sh.at[pl.ds(s * rows_per_tile, rows_per_tile)],
                        out_hbm.at[c].at[pl.ds(s * rows_per_tile, rows_per_tile)])

    return sc_aggregate


def _tc_dense(parts, h, w_rel_t, w_root_t, b_2d):
    n_pad, d = h.shape
    blk = 1024

    def body(p_ref, h_ref, wr_ref, wo_ref, b_ref, o_ref):
        aggr = p_ref[0] + p_ref[1]
        z = jnp.dot(aggr, wr_ref[...], preferred_element_type=jnp.float32)
        z = z + jnp.dot(h_ref[...], wo_ref[...], preferred_element_type=jnp.float32)
        z = z + b_ref[...]
        o_ref[...] = jnp.where(z > 0, z, jnp.exp(jnp.minimum(z, 0.0)) - 1.0)

    return pl.pallas_call(
        body,
        grid=(n_pad // blk,),
        in_specs=[
            pl.BlockSpec((_NC, blk, d), lambda i: (0, i, 0)),
            pl.BlockSpec((blk, d), lambda i: (i, 0)),
            pl.BlockSpec((d, d), lambda i: (0, 0)),
            pl.BlockSpec((d, d), lambda i: (0, 0)),
            pl.BlockSpec((1, d), lambda i: (0, 0)),
        ],
        out_specs=pl.BlockSpec((blk, d), lambda i: (i, 0)),
        out_shape=jax.ShapeDtypeStruct((n_pad, d), jnp.float32),
    )(parts, h, w_rel_t, w_root_t, b_2d)


def kernel(x, edge_index, W1_rel, b1_rel, W1_root, W2_rel, b2_rel, W2_root,
           W3_rel, b3_rel, W3_root):
    n, d = x.shape
    e = edge_index.shape[1]

    cpt = -(-e // (_NW * _CH * 4)) * 4
    e_pad = cpt * _NW * _CH
    n_pad = -(-(n + 1) // (_NS * _CH)) * (_NS * _CH)

    ei = edge_index.astype(jnp.int32)
    fill = n + jnp.arange(e_pad - e, dtype=jnp.int32) % (n_pad - n)
    src2d = jnp.concatenate([ei[0], fill]).reshape(-1, _CH)
    dst2d = jnp.concatenate([ei[1], fill]).reshape(-1, _CH)

    h = jnp.zeros((n_pad, d), jnp.float32).at[:n].set(x)
    sc_aggregate = _make_sc_aggregate(n_pad, cpt, d)

    for w_rel, b_rel, w_root in ((W1_rel, b1_rel, W1_root),
                                 (W2_rel, b2_rel, W2_root),
                                 (W3_rel, b3_rel, W3_root)):
        parts = sc_aggregate(h, src2d, dst2d)
        h = _tc_dense(parts, h, w_rel.T, w_root.T, b_rel.reshape(1, d))
    return h[:n]


# speedup vs baseline: 9.8901x; 1.0321x over previous
import functools

import jax
import jax.numpy as jnp
from jax import lax
from jax.experimental import pallas as pl
from jax.experimental.pallas import tpu as pltpu
from jax.experimental.pallas import tpu_sc as plsc

_CH = 128
_NC = 2
_NS = 16
_NW = _NC * _NS


@functools.lru_cache(maxsize=None)
def _make_sc_aggregate(n_pad: int, cpt: int, d: int):
    rows_per_tile = n_pad // _NS
    mesh = plsc.VectorSubcoreMesh(core_axis_name="c", subcore_axis_name="s")

    @functools.partial(
        pl.kernel,
        out_type=jax.ShapeDtypeStruct((_NC, n_pad, d), jnp.float32),
        mesh=mesh,
        scratch_types=[
            pltpu.VMEM((cpt // 2, _CH), jnp.int32),
            pltpu.VMEM((cpt // 2, _CH), jnp.int32),
            pltpu.VMEM((_CH, d), jnp.float32),
            pltpu.VMEM((_CH, d), jnp.float32),
            pltpu.VMEM_SHARED((n_pad, d), jnp.float32),
            pltpu.SemaphoreType.DMA,
            pltpu.SemaphoreType.DMA,
            pltpu.SemaphoreType.DMA,
            pltpu.SemaphoreType.DMA,
        ],
    )
    def sc_aggregate(h_hbm, src_hbm, dst_hbm, out_hbm,
                     src_v, dst_v, buf_a, buf_b, aggr_sh,
                     sem_a, sem_b, sem_sa, sem_sb):
        c = lax.axis_index("c")
        s = lax.axis_index("s")
        wid = s * _NC + c

        zero16 = jnp.zeros((16,), jnp.float32)

        def zero_row(i, carry):
            for j in range(d // 16):
                buf_a[i, pl.ds(j * 16, 16)] = zero16
            return carry

        lax.fori_loop(0, _CH, zero_row, 0)
        for k in range(rows_per_tile // _CH):
            pltpu.sync_copy(buf_a, aggr_sh.at[pl.ds(s * rows_per_tile + k * _CH, _CH)])
        plsc.subcore_barrier()

        cpp = cpt // 2

        def gather(j, buf, sem):
            return pltpu.async_copy(h_hbm.at[src_v.at[j]], buf, sem)

        def scatter(j, buf, sem):
            return pltpu.async_copy(buf, aggr_sh.at[dst_v.at[j]], sem, add=True)

        def wait_gather(j, buf, sem):
            pltpu.make_async_copy(h_hbm.at[src_v.at[j]], buf, sem).wait()

        def body(i, carry):
            j0 = 2 * i
            j1 = j0 + 1
            wait_gather(j0, buf_a, sem_a)
            sc_a = scatter(j0, buf_a, sem_sa)
            wait_gather(j1, buf_b, sem_b)
            sc_b = scatter(j1, buf_b, sem_sb)
            sc_a.wait()

            @pl.when(j0 + 2 < cpp)
            def _():
                gather(j0 + 2, buf_a, sem_a)
            sc_b.wait()

            @pl.when(j1 + 2 < cpp)
            def _():
                gather(j1 + 2, buf_b, sem_b)
            return carry

        for phase in range(2):
            pltpu.sync_copy(src_hbm.at[pl.ds(wid * cpt + phase * cpp, cpp)], src_v)
            pltpu.sync_copy(dst_hbm.at[pl.ds(wid * cpt + phase * cpp, cpp)], dst_v)
            gather(0, buf_a, sem_a)
            gather(1, buf_b, sem_b)
            lax.fori_loop(0, cpp // 2, body, 0)
        plsc.subcore_barrier()

        pltpu.sync_copy(aggr_sh.at[pl.ds(s * rows_per_tile, rows_per_tile)],
                        out_hbm.at[c].at[pl.ds(s * rows_per_tile, rows_per_tile)])

    return sc_aggregate


def _tc_dense(parts, h, w_rel_t, w_root_t, b_2d):
    n_pad, d = h.shape
    blk = 1024

    def body(p_ref, h_ref, wr_ref, wo_ref, b_ref, o_ref):
        aggr = p_ref[0] + p_ref[1]
        z = jnp.dot(aggr, wr_ref[...], preferred_element_type=jnp.float32)
        z = z + jnp.dot(h_ref[...], wo_ref[...], preferred_element_type=jnp.float32)
        z = z + b_ref[...]
        o_ref[...] = jnp.where(z > 0, z, jnp.exp(jnp.minimum(z, 0.0)) - 1.0)

    return pl.pallas_call(
        body,
        grid=(n_pad // blk,),
        in_specs=[
            pl.BlockSpec((_NC, blk, d), lambda i: (0, i, 0)),
            pl.BlockSpec((blk, d), lambda i: (i, 0)),
            pl.BlockSpec((d, d), lambda i: (0, 0)),
            pl.BlockSpec((d, d), lambda i: (0, 0)),
            pl.BlockSpec((1, d), lambda i: (0, 0)),
        ],
        out_specs=pl.BlockSpec((blk, d), lambda i: (i, 0)),
        out_shape=jax.ShapeDtypeStruct((n_pad, d), jnp.float32),
    )(parts, h, w_rel_t, w_root_t, b_2d)


def kernel(x, edge_index, W1_rel, b1_rel, W1_root, W2_rel, b2_rel, W2_root,
           W3_rel, b3_rel, W3_root):
    n, d = x.shape
    e = edge_index.shape[1]

    cpt = -(-e // (_NW * _CH * 4)) * 4
    e_pad = cpt * _NW * _CH
    n_pad = -(-(n + 1) // (_NS * _CH)) * (_NS * _CH)

    ei = edge_index.astype(jnp.int32)
    fill = n + jnp.arange(e_pad - e, dtype=jnp.int32) % (n_pad - n)
    src2d = jnp.concatenate([ei[0], fill]).reshape(-1, _CH)
    dst2d = jnp.concatenate([ei[1], fill]).reshape(-1, _CH)

    h = jnp.zeros((n_pad, d), jnp.float32).at[:n].set(x)
    sc_aggregate = _make_sc_aggregate(n_pad, cpt, d)

    for w_rel, b_rel, w_root in ((W1_rel, b1_rel, W1_root),
                                 (W2_rel, b2_rel, W2_root),
                                 (W3_rel, b3_rel, W3_root)):
        parts = sc_aggregate(h, src2d, dst2d)
        h = _tc_dense(parts, h, w_rel.T, w_root.T, b_rel.reshape(1, d))
    return h[:n]


# speedup vs baseline: 12.0553x; 1.2189x over previous
import functools

import jax
import jax.numpy as jnp
from jax import lax
from jax.experimental import pallas as pl
from jax.experimental.pallas import tpu as pltpu
from jax.experimental.pallas import tpu_sc as plsc

_CH = 64
_NB = 4
_NP = 4
_NC = 2
_NS = 16
_NW = _NC * _NS


@functools.lru_cache(maxsize=None)
def _make_sc_aggregate(n_pad: int, cpt: int, d: int):
    rows_per_tile = n_pad // _NS
    mesh = plsc.VectorSubcoreMesh(core_axis_name="c", subcore_axis_name="s")

    @functools.partial(
        pl.kernel,
        out_type=jax.ShapeDtypeStruct((_NC, n_pad, d), jnp.float32),
        mesh=mesh,
        scratch_types=[
            pltpu.VMEM((cpt // _NP, _CH), jnp.int32),
            pltpu.VMEM((cpt // _NP, _CH), jnp.int32),
            [pltpu.VMEM((_CH, d), jnp.float32) for _ in range(_NB)],
            pltpu.VMEM((40, d), jnp.float32),
            pltpu.VMEM_SHARED((n_pad, d), jnp.float32),
            [pltpu.SemaphoreType.DMA for _ in range(_NB)],
            [pltpu.SemaphoreType.DMA for _ in range(_NB)],
            pltpu.SemaphoreType.DMA,
        ],
    )
    def sc_aggregate(h_hbm, src_hbm, dst_hbm, out_hbm,
                     src_v, dst_v, bufs, zbuf, aggr_sh, sem_g, sem_s, sem_z):
        c = lax.axis_index("c")
        s = lax.axis_index("s")
        wid = s * _NC + c

        cpp = cpt // _NP

        def gather(j, buf, sem):
            return pltpu.async_copy(h_hbm.at[src_v.at[j]], buf, sem)

        def scatter(j, buf, sem):
            return pltpu.async_copy(buf, aggr_sh.at[dst_v.at[j]], sem, add=True)

        def wait_gather(j, buf, sem):
            pltpu.make_async_copy(h_hbm.at[src_v.at[j]], buf, sem).wait()

        def body(i, carry):
            scs = []
            for u in range(_NB):
                j = _NB * i + u
                wait_gather(j, bufs[u], sem_g[u])
                scs.append(scatter(j, bufs[u], sem_s[u]))
            for u in range(_NB):
                j = _NB * i + u
                scs[u].wait()

                @pl.when(j + _NB < cpp)
                def _():
                    gather(j + _NB, bufs[u], sem_g[u])
            return carry

        def stage_idx(phase):
            base = wid * cpt + phase * cpp
            ca = pltpu.async_copy(src_hbm.at[pl.ds(base, cpp)], src_v, sem_z)
            cb = pltpu.async_copy(dst_hbm.at[pl.ds(base, cpp)], dst_v, sem_z)
            ca.wait()
            cb.wait()

        stage_idx(0)
        for u in range(_NB):
            gather(u, bufs[u], sem_g[u])

        zero16 = jnp.zeros((16,), jnp.float32)

        def zero_row(i, carry):
            for j in range(d // 16):
                zbuf[i, pl.ds(j * 16, 16)] = zero16
            return carry

        lax.fori_loop(0, 40, zero_row, 0)
        zcs = [pltpu.async_copy(zbuf,
                                aggr_sh.at[pl.ds(s * rows_per_tile + k * 40, 40)],
                                sem_z)
               for k in range(rows_per_tile // 40)]
        for zc in zcs:
            zc.wait()
        plsc.subcore_barrier()

        for phase in range(_NP):
            if phase:
                stage_idx(phase)
                for u in range(_NB):
                    gather(u, bufs[u], sem_g[u])
            lax.fori_loop(0, cpp // _NB, body, 0)
        plsc.subcore_barrier()

        pltpu.sync_copy(aggr_sh.at[pl.ds(s * rows_per_tile, rows_per_tile)],
                        out_hbm.at[c].at[pl.ds(s * rows_per_tile, rows_per_tile)])

    return sc_aggregate


def _tc_dense(parts, h, w_rel_t, w_root_t, b_2d):
    n_pad, d = h.shape
    blk = 2048

    def body(p_ref, h_ref, wr_ref, wo_ref, b_ref, o_ref):
        aggr = p_ref[0] + p_ref[1]
        z = jnp.dot(aggr, wr_ref[...], preferred_element_type=jnp.float32)
        z = z + jnp.dot(h_ref[...], wo_ref[...], preferred_element_type=jnp.float32)
        z = z + b_ref[...]
        o_ref[...] = jnp.where(z > 0, z, jnp.exp(jnp.minimum(z, 0.0)) - 1.0)

    return pl.pallas_call(
        body,
        grid=(n_pad // blk,),
        in_specs=[
            pl.BlockSpec((_NC, blk, d), lambda i: (0, i, 0)),
            pl.BlockSpec((blk, d), lambda i: (i, 0)),
            pl.BlockSpec((d, d), lambda i: (0, 0)),
            pl.BlockSpec((d, d), lambda i: (0, 0)),
            pl.BlockSpec((1, d), lambda i: (0, 0)),
        ],
        out_specs=pl.BlockSpec((blk, d), lambda i: (i, 0)),
        out_shape=jax.ShapeDtypeStruct((n_pad, d), jnp.float32),
    )(parts, h, w_rel_t, w_root_t, b_2d)


def kernel(x, edge_index, W1_rel, b1_rel, W1_root, W2_rel, b2_rel, W2_root,
           W3_rel, b3_rel, W3_root):
    n, d = x.shape
    e = edge_index.shape[1]

    cpt = -(-e // (_NW * _CH * _NP * _NB)) * (_NP * _NB)
    e_pad = cpt * _NW * _CH
    n_pad = -(-(n + 1) // (_NS * _CH)) * (_NS * _CH)

    ei = edge_index.astype(jnp.int32)
    fill = n + jnp.arange(e_pad - e, dtype=jnp.int32) % (n_pad - n)
    src2d = jnp.concatenate([ei[0], fill]).reshape(-1, _CH)
    dst2d = jnp.concatenate([ei[1], fill]).reshape(-1, _CH)

    h = jnp.zeros((n_pad, d), jnp.float32).at[:n].set(x)
    sc_aggregate = _make_sc_aggregate(n_pad, cpt, d)

    for w_rel, b_rel, w_root in ((W1_rel, b1_rel, W1_root),
                                 (W2_rel, b2_rel, W2_root),
                                 (W3_rel, b3_rel, W3_root)):
        parts = sc_aggregate(h, src2d, dst2d)
        h = _tc_dense(parts, h, w_rel.T, w_root.T, b_rel.reshape(1, d))
    return h[:n]
